# baseline (speedup 1.0000x reference)
import jax
import jax.numpy as jnp
from jax.experimental import pallas as pl
from jax.experimental.pallas import tpu as pltpu

N, D, C, A, H = 50000, 2048, 50, 512, 1024
K = 100
BM = 1024


def _rownorm_sq(x, bm):
    x2 = x * x
    chunks = [x2[:, k * 128:(k + 1) * 128] for k in range(16)]

    def chain(cs):
        acc = cs[0] + cs[1]
        for c in cs[2:]:
            acc = acc + c
        return acc

    a = chain(chunks[:8])
    b = chain(chunks[8:])
    g = bm // 128

    def tsum(m):
        mt = jnp.swapaxes(m.reshape(g, 128, 128), 1, 2)
        acc = mt[:, 0:8, :]
        for k in range(1, 16):
            acc = acc + mt[:, 8 * k:8 * (k + 1), :]
        return acc

    s = tsum(a) + tsum(b)
    s0, s1, s2, s3 = s[:, 0, :], s[:, 1, :], s[:, 2, :], s[:, 3, :]
    s4, s5, s6, s7 = s[:, 4, :], s[:, 5, :], s[:, 6, :], s[:, 7, :]
    tot = ((s0 + s4) + (s2 + s6)) + ((s1 + s5) + (s3 + s7))
    bc = jnp.broadcast_to(tot.reshape(g, 1, 128), (g, 128, 128))
    col = jnp.swapaxes(bc, 1, 2)[:, :, 0:1]
    return col.reshape(bm, 1)


def _proto_kernel(attr_ref, w1_ref, b1_ref, w2_ref, b2_ref, proto_ref):
    h = jnp.maximum(jnp.dot(attr_ref[...], w1_ref[...],
                            preferred_element_type=jnp.float32) + b1_ref[...], 0.0)
    p = jnp.maximum(jnp.dot(h, w2_ref[...],
                            preferred_element_type=jnp.float32) + b2_ref[...], 0.0)
    p_pad = jnp.concatenate([p, jnp.zeros((128 - C, D), jnp.float32)], axis=0)
    nrm = jnp.sqrt(_rownorm_sq(p_pad, 128)[:C])
    proto_ref[...] = p / (nrm + 1e-12)


def _scores_kernel(x_ref, protot_ref, scores_ref, maxval_ref, pred_ref):
    x = x_ref[...]
    nrm = jnp.sqrt(_rownorm_sq(x, x.shape[0]))
    feat = x / (nrm + 1e-12)
    s = jnp.dot(feat, protot_ref[...], preferred_element_type=jnp.float32)
    scores_ref[...] = s
    maxval_ref[...] = jnp.max(s, axis=-1, keepdims=True)
    pred_ref[...] = jnp.argmax(s, axis=-1, keepdims=True).astype(jnp.int32)


def _gather_kernel(idx_ref, x_ref, o_ref):
    o_ref[...] = x_ref[...]


def kernel(test_unseen_feat, unseen_attr, fc1_w, fc1_b, fc2_w, fc2_b):
    proto = pl.pallas_call(
        _proto_kernel,
        out_shape=jax.ShapeDtypeStruct((C, D), jnp.float32),
    )(unseen_attr, fc1_w, fc1_b.reshape(1, H), fc2_w, fc2_b.reshape(1, D))

    protot = proto.T

    grid = (pl.cdiv(N, BM),)
    scores, max_val, pred_idx = pl.pallas_call(
        _scores_kernel,
        grid=grid,
        in_specs=[
            pl.BlockSpec((BM, D), lambda i: (i, 0)),
            pl.BlockSpec((D, C), lambda i: (0, 0)),
        ],
        out_specs=[
            pl.BlockSpec((BM, C), lambda i: (i, 0)),
            pl.BlockSpec((BM, 1), lambda i: (i, 0)),
            pl.BlockSpec((BM, 1), lambda i: (i, 0)),
        ],
        out_shape=[
            jax.ShapeDtypeStruct((N, C), jnp.float32),
            jax.ShapeDtypeStruct((N, 1), jnp.float32),
            jax.ShapeDtypeStruct((N, 1), jnp.int32),
        ],
    )(test_unseen_feat, protot)

    max_val = max_val.reshape(N)
    pred_idx = pred_idx.reshape(N)

    top_vals, top_idx = jax.lax.top_k(scores.T, K)
    selected_idx = top_idx.T.reshape(-1)
    outpred = jnp.tile(jnp.arange(C, dtype=jnp.int32), K)

    x3 = test_unseen_feat.reshape(N, 16, 128)
    selected_feat = pl.pallas_call(
        _gather_kernel,
        grid_spec=pltpu.PrefetchScalarGridSpec(
            num_scalar_prefetch=1,
            grid=(K * C,),
            in_specs=[pl.BlockSpec((1, 16, 128), lambda i, idx_ref: (idx_ref[i], 0, 0))],
            out_specs=pl.BlockSpec((1, 16, 128), lambda i, idx_ref: (i, 0, 0)),
        ),
        out_shape=jax.ShapeDtypeStruct((K * C, 16, 128), jnp.float32),
    )(selected_idx, x3).reshape(K * C, D)

    return scores, selected_feat, pred_idx, max_val, selected_idx, outpred


# speedup vs baseline: 2.3210x; 2.3210x over previous
import jax
import jax.numpy as jnp
from jax.experimental import pallas as pl
from jax.experimental.pallas import tpu as pltpu

N, D, C, A, H = 50000, 2048, 50, 512, 1024
K = 100
BM = 1024


def _rownorm_sq(x, bm):
    x2 = x * x
    chunks = [x2[:, k * 128:(k + 1) * 128] for k in range(16)]

    def chain(cs):
        acc = cs[0] + cs[1]
        for c in cs[2:]:
            acc = acc + c
        return acc

    a = chain(chunks[:8])
    b = chain(chunks[8:])
    g = bm // 128

    def tsum(m):
        mt = jnp.swapaxes(m.reshape(g, 128, 128), 1, 2)
        acc = mt[:, 0:8, :]
        for k in range(1, 16):
            acc = acc + mt[:, 8 * k:8 * (k + 1), :]
        return acc

    s = tsum(a) + tsum(b)
    s0, s1, s2, s3 = s[:, 0, :], s[:, 1, :], s[:, 2, :], s[:, 3, :]
    s4, s5, s6, s7 = s[:, 4, :], s[:, 5, :], s[:, 6, :], s[:, 7, :]
    tot = ((s0 + s4) + (s2 + s6)) + ((s1 + s5) + (s3 + s7))
    bc = jnp.broadcast_to(tot.reshape(g, 1, 128), (g, 128, 128))
    col = jnp.swapaxes(bc, 1, 2)[:, :, 0:1]
    return col.reshape(bm, 1)


def _proto_kernel(attr_ref, w1_ref, b1_ref, w2_ref, b2_ref, proto_ref):
    h = jnp.maximum(jnp.dot(attr_ref[...], w1_ref[...],
                            preferred_element_type=jnp.float32) + b1_ref[...], 0.0)
    p = jnp.maximum(jnp.dot(h, w2_ref[...],
                            preferred_element_type=jnp.float32) + b2_ref[...], 0.0)
    p_pad = jnp.concatenate([p, jnp.zeros((128 - C, D), jnp.float32)], axis=0)
    nrm = jnp.sqrt(_rownorm_sq(p_pad, 128)[:C])
    proto_ref[...] = p / (nrm + 1e-12)


def _scores_kernel(x_ref, protot_ref, scores_ref, maxval_ref, pred_ref):
    x = x_ref[...]
    nrm = jnp.sqrt(_rownorm_sq(x, x.shape[0]))
    feat = x / (nrm + 1e-12)
    s = jnp.dot(feat, protot_ref[...], preferred_element_type=jnp.float32)
    scores_ref[...] = s
    maxval_ref[...] = jnp.max(s, axis=-1, keepdims=True)
    pred_ref[...] = jnp.argmax(s, axis=-1, keepdims=True).astype(jnp.int32)


RG = 40


def _gather_kernel(idx_ref, x_ref, o_ref, sem):
    i = pl.program_id(0)
    copies = [
        pltpu.make_async_copy(
            x_ref.at[pl.ds(idx_ref[i * RG + j], 1), :],
            o_ref.at[pl.ds(j, 1), :],
            sem,
        )
        for j in range(RG)
    ]
    for c in copies:
        c.start()
    for c in copies:
        c.wait()


def kernel(test_unseen_feat, unseen_attr, fc1_w, fc1_b, fc2_w, fc2_b):
    proto = pl.pallas_call(
        _proto_kernel,
        out_shape=jax.ShapeDtypeStruct((C, D), jnp.float32),
    )(unseen_attr, fc1_w, fc1_b.reshape(1, H), fc2_w, fc2_b.reshape(1, D))

    protot = proto.T

    grid = (pl.cdiv(N, BM),)
    scores, max_val, pred_idx = pl.pallas_call(
        _scores_kernel,
        grid=grid,
        in_specs=[
            pl.BlockSpec((BM, D), lambda i: (i, 0)),
            pl.BlockSpec((D, C), lambda i: (0, 0)),
        ],
        out_specs=[
            pl.BlockSpec((BM, C), lambda i: (i, 0)),
            pl.BlockSpec((BM, 1), lambda i: (i, 0)),
            pl.BlockSpec((BM, 1), lambda i: (i, 0)),
        ],
        out_shape=[
            jax.ShapeDtypeStruct((N, C), jnp.float32),
            jax.ShapeDtypeStruct((N, 1), jnp.float32),
            jax.ShapeDtypeStruct((N, 1), jnp.int32),
        ],
    )(test_unseen_feat, protot)

    max_val = max_val.reshape(N)
    pred_idx = pred_idx.reshape(N)

    top_vals, top_idx = jax.lax.top_k(scores.T, K)
    selected_idx = top_idx.T.reshape(-1)
    outpred = jnp.tile(jnp.arange(C, dtype=jnp.int32), K)

    selected_feat = pl.pallas_call(
        _gather_kernel,
        grid_spec=pltpu.PrefetchScalarGridSpec(
            num_scalar_prefetch=1,
            grid=(K * C // RG,),
            in_specs=[pl.BlockSpec(memory_space=pl.ANY)],
            out_specs=pl.BlockSpec((RG, D), lambda i, idx_ref: (i, 0)),
            scratch_shapes=[pltpu.SemaphoreType.DMA],
        ),
        out_shape=jax.ShapeDtypeStruct((K * C, D), jnp.float32),
    )(selected_idx, test_unseen_feat)

    return scores, selected_feat, pred_idx, max_val, selected_idx, outpred


# speedup vs baseline: 4.3961x; 1.8941x over previous
import jax
import jax.numpy as jnp
from jax.experimental import pallas as pl
from jax.experimental.pallas import tpu as pltpu

N, D, C, A, H = 50000, 2048, 50, 512, 1024
K = 100
BM = 1024


def _rownorm_sq(x, bm):
    x2 = x * x
    chunks = [x2[:, k * 128:(k + 1) * 128] for k in range(16)]

    def chain(cs):
        acc = cs[0] + cs[1]
        for c in cs[2:]:
            acc = acc + c
        return acc

    a = chain(chunks[:8])
    b = chain(chunks[8:])
    g = bm // 128

    def tsum(m):
        mt = jnp.swapaxes(m.reshape(g, 128, 128), 1, 2)
        acc = mt[:, 0:8, :]
        for k in range(1, 16):
            acc = acc + mt[:, 8 * k:8 * (k + 1), :]
        return acc

    s = tsum(a) + tsum(b)
    s0, s1, s2, s3 = s[:, 0, :], s[:, 1, :], s[:, 2, :], s[:, 3, :]
    s4, s5, s6, s7 = s[:, 4, :], s[:, 5, :], s[:, 6, :], s[:, 7, :]
    tot = ((s0 + s4) + (s2 + s6)) + ((s1 + s5) + (s3 + s7))
    bc = jnp.broadcast_to(tot.reshape(g, 1, 128), (g, 128, 128))
    col = jnp.swapaxes(bc, 1, 2)[:, :, 0:1]
    return col.reshape(bm, 1)


def _proto_kernel(attr_ref, w1_ref, b1_ref, w2_ref, b2_ref, proto_ref):
    h = jnp.maximum(jnp.dot(attr_ref[...], w1_ref[...],
                            preferred_element_type=jnp.float32) + b1_ref[...], 0.0)
    p = jnp.maximum(jnp.dot(h, w2_ref[...],
                            preferred_element_type=jnp.float32) + b2_ref[...], 0.0)
    p_pad = jnp.concatenate([p, jnp.zeros((128 - C, D), jnp.float32)], axis=0)
    nrm = jnp.sqrt(_rownorm_sq(p_pad, 128)[:C])
    proto_ref[...] = p / (nrm + 1e-12)


def _scores_kernel(x_ref, protot_ref, scores_ref, maxval_ref, pred_ref):
    x = x_ref[...]
    nrm = jnp.sqrt(_rownorm_sq(x, x.shape[0]))
    feat = x / (nrm + 1e-12)
    s = jnp.dot(feat, protot_ref[...], preferred_element_type=jnp.float32)
    scores_ref[...] = s
    maxval_ref[...] = jnp.max(s, axis=-1, keepdims=True)
    pred_ref[...] = jnp.argmax(s, axis=-1, keepdims=True).astype(jnp.int32)


def _topk_kernel(st_ref, idx_ref):
    s = st_ref[...]
    iota = jax.lax.broadcasted_iota(jnp.int32, (C, N), 1)
    klane = jax.lax.broadcasted_iota(jnp.int32, (C, 128), 1)

    def body(k, carry):
        s, iacc = carry
        m = jnp.max(s, axis=1, keepdims=True)
        cand = jnp.where(s == m, iota, N)
        idx = jnp.min(cand, axis=1, keepdims=True)
        s = jnp.where(cand == idx, -jnp.inf, s)
        iacc = jnp.where(klane == k, idx, iacc)
        return s, iacc

    _, iacc = jax.lax.fori_loop(
        0, K, body, (s, jnp.zeros((C, 128), jnp.int32)))
    idx_ref[...] = iacc


RG = 40


def _gather_kernel(idx_ref, x_ref, o_ref, sem):
    i = pl.program_id(0)
    copies = [
        pltpu.make_async_copy(
            x_ref.at[pl.ds(idx_ref[i * RG + j], 1), :],
            o_ref.at[pl.ds(j, 1), :],
            sem,
        )
        for j in range(RG)
    ]
    for c in copies:
        c.start()
    for c in copies:
        c.wait()


def kernel(test_unseen_feat, unseen_attr, fc1_w, fc1_b, fc2_w, fc2_b):
    proto = pl.pallas_call(
        _proto_kernel,
        out_shape=jax.ShapeDtypeStruct((C, D), jnp.float32),
    )(unseen_attr, fc1_w, fc1_b.reshape(1, H), fc2_w, fc2_b.reshape(1, D))

    protot = proto.T

    grid = (pl.cdiv(N, BM),)
    scores, max_val, pred_idx = pl.pallas_call(
        _scores_kernel,
        grid=grid,
        in_specs=[
            pl.BlockSpec((BM, D), lambda i: (i, 0)),
            pl.BlockSpec((D, C), lambda i: (0, 0)),
        ],
        out_specs=[
            pl.BlockSpec((BM, C), lambda i: (i, 0)),
            pl.BlockSpec((BM, 1), lambda i: (i, 0)),
            pl.BlockSpec((BM, 1), lambda i: (i, 0)),
        ],
        out_shape=[
            jax.ShapeDtypeStruct((N, C), jnp.float32),
            jax.ShapeDtypeStruct((N, 1), jnp.float32),
            jax.ShapeDtypeStruct((N, 1), jnp.int32),
        ],
    )(test_unseen_feat, protot)

    max_val = max_val.reshape(N)
    pred_idx = pred_idx.reshape(N)

    top_idx = pl.pallas_call(
        _topk_kernel,
        out_shape=jax.ShapeDtypeStruct((C, 128), jnp.int32),
    )(scores.T)[:, :K]
    selected_idx = top_idx.T.reshape(-1)
    outpred = jnp.tile(jnp.arange(C, dtype=jnp.int32), K)

    selected_feat = pl.pallas_call(
        _gather_kernel,
        grid_spec=pltpu.PrefetchScalarGridSpec(
            num_scalar_prefetch=1,
            grid=(K * C // RG,),
            in_specs=[pl.BlockSpec(memory_space=pl.ANY)],
            out_specs=pl.BlockSpec((RG, D), lambda i, idx_ref: (i, 0)),
            scratch_shapes=[pltpu.SemaphoreType.DMA],
        ),
        out_shape=jax.ShapeDtypeStruct((K * C, D), jnp.float32),
    )(selected_idx, test_unseen_feat)

    return scores, selected_feat, pred_idx, max_val, selected_idx, outpred
